# Initial kernel scaffold; baseline (speedup 1.0000x reference)
import jax
import jax.numpy as jnp
from jax.experimental import pallas as pl


def kernel(encoded_words, embed_table):
    raise NotImplementedError("write your pallas kernel here")


# speedup vs baseline: 2.6064x; 2.6064x over previous
import functools
import math

import jax
import jax.numpy as jnp
import numpy as np
from jax import lax
from jax.experimental import pallas as pl
from jax.experimental.pallas import tpu as pltpu
from jax.experimental.pallas import tpu_sc as plsc

VOCAB = 100000
D_MODEL = 128
MAX_LEN = 50
BATCH = 4096
SEQ = 50

N_TOKENS = BATCH * SEQ
CHUNK = 200
GATHER = 100
SCALE = math.sqrt(D_MODEL)


def _make_pe_tile():
    pe = np.zeros((MAX_LEN, D_MODEL), dtype=np.float32)
    position = np.arange(MAX_LEN, dtype=np.float32)[:, None]
    div_term = np.exp(
        np.arange(0, D_MODEL, 2, dtype=np.float32) * -(math.log(10000.0) / D_MODEL)
    )
    pe[:, 0::2] = np.sin(position * div_term)
    pe[:, 1::2] = np.cos(position * div_term)
    reps = CHUNK // MAX_LEN
    return np.tile(pe, (reps, 1))


_PE_TILE = _make_pe_tile()


def kernel(encoded_words, embed_table):
    info = plsc.get_sparse_core_info()
    nw = info.num_cores * info.num_subcores
    b_per_w = N_TOKENS // nw
    n_chunks = b_per_w // CHUNK

    n_g = CHUNK // GATHER
    idx = encoded_words.astype(jnp.int32).reshape(nw, n_chunks, n_g, GATHER)
    pe_tile = jnp.asarray(_PE_TILE)

    mesh = plsc.VectorSubcoreMesh(core_axis_name="c", subcore_axis_name="s")

    @functools.partial(
        pl.kernel,
        mesh=mesh,
        out_type=jax.ShapeDtypeStruct((N_TOKENS, D_MODEL), jnp.float32),
        scratch_types=[
            pltpu.VMEM((n_chunks, n_g, GATHER), jnp.int32),
            pltpu.VMEM((CHUNK, D_MODEL), jnp.float32),
            pltpu.VMEM((CHUNK, D_MODEL), jnp.float32),
            pltpu.SemaphoreType.DMA,
        ],
    )
    def run(table_hbm, idx_hbm, pe_hbm, out_hbm, idx_v, buf, pe_v, sem):
        wid = lax.axis_index("s") * info.num_cores + lax.axis_index("c")
        base = wid * b_per_w
        pltpu.sync_copy(idx_hbm.at[wid], idx_v)
        pltpu.sync_copy(pe_hbm, pe_v)

        def chunk_body(j, _):
            for g in range(n_g):
                pltpu.async_copy(
                    table_hbm.at[idx_v.at[j, g]],
                    buf.at[pl.ds(g * GATHER, GATHER)],
                    sem,
                )
            for g in range(n_g):
                pltpu.make_async_copy(
                    table_hbm.at[idx_v.at[j, g]],
                    buf.at[pl.ds(g * GATHER, GATHER)],
                    sem,
                ).wait()

            def row_body(r, _):
                for c in range(D_MODEL // 16):
                    sl = pl.ds(c * 16, 16)
                    buf[r, sl] = buf[r, sl] * SCALE + pe_v[r, sl]
                return 0

            lax.fori_loop(0, CHUNK, row_body, 0)
            pltpu.sync_copy(buf, out_hbm.at[pl.ds(base + j * CHUNK, CHUNK)])
            return 0

        lax.fori_loop(0, n_chunks, chunk_body, 0)

    out = run(embed_table, idx, pe_tile)
    return out.reshape(BATCH, SEQ, D_MODEL)
